# baseline (speedup 1.0000x reference)
import functools

import jax
import jax.numpy as jnp
from jax import lax
from jax.experimental import pallas as pl
from jax.experimental.pallas import tpu as pltpu
from jax.experimental.pallas import tpu_sc as plsc

_ED = 64
_LD = 64
_GD = 2 * _ED
_S = 16384
_NB = 31
_D = _NB * _S
_V = 1000000
_CHUNK = 128
_NC = 2
_NS = 16
_NW = _NC * _NS


def _project_body(alo_ref, ahi_ref, wd_ref, o_ref):
  i = pl.program_id(0)
  lane_g = lax.broadcasted_iota(jnp.int32, (_ED, _S), 1) + i * _S
  ahi = jnp.where(lane_g >= _V - _D, 0.0, ahi_ref[...])
  a2 = jnp.concatenate([alo_ref[...], ahi], axis=0)
  dn = (((0,), (0,)), ((), ()))
  o_ref[...] = lax.dot_general(a2, wd_ref[...], dn,
                               preferred_element_type=jnp.float32)


def _project(table_t, wd):
  return pl.pallas_call(
      _project_body,
      grid=(_NB,),
      in_specs=[
          pl.BlockSpec((_ED, _S), lambda i: (0, i)),
          pl.BlockSpec((_ED, _S), lambda i: (0, jnp.minimum(i + _NB, _V // _S))),
          pl.BlockSpec((_GD, _GD), lambda i: (0, 0)),
      ],
      out_specs=pl.BlockSpec((_S, _GD), lambda i: (i, 0)),
      out_shape=jax.ShapeDtypeStruct((_D, _GD), jnp.float32),
  )(table_t, table_t, wd)


def _gather_body(nch, chunk, table_hbm, idx_hbm, out_hbm, idx_v, rows_v,
                 sem0, sem1):
  wid = lax.axis_index("s") * _NC + lax.axis_index("c")
  base = wid * (nch * chunk)
  pltpu.sync_copy(idx_hbm.at[wid], idx_v)

  sems = (sem0, sem1)

  def start(c, buf):
    pltpu.make_async_copy(
        table_hbm.at[idx_v.at[pl.ds(c * chunk, chunk)]],
        rows_v.at[buf], sems[buf]).start()

  def wait(c, buf):
    pltpu.make_async_copy(
        table_hbm.at[idx_v.at[pl.ds(c * chunk, chunk)]],
        rows_v.at[buf], sems[buf]).wait()

  def store(c, buf):
    pltpu.sync_copy(rows_v.at[buf], out_hbm.at[pl.ds(base + c * chunk, chunk)])

  start(0, 0)

  def body(g, carry):
    c = 2 * g
    start(c + 1, 1)
    wait(c, 0)
    store(c, 0)

    @pl.when(c + 2 < nch)
    def _():
      start(c + 2, 0)

    wait(c + 1, 1)
    store(c + 1, 1)
    return carry

  lax.fori_loop(0, nch // 2, body, 0)
  if nch % 2 == 1:
    wait(nch - 1, 0)
    store(nch - 1, 0)


def _sc_gather(p2, ids2, chunk):
  nw, nper = ids2.shape
  nch = nper // chunk
  n = nw * nper
  mesh = plsc.VectorSubcoreMesh(core_axis_name="c", subcore_axis_name="s")
  f = functools.partial(
      pl.kernel,
      mesh=mesh,
      out_type=jax.ShapeDtypeStruct((n, _GD), jnp.float32),
      scratch_types=[
          pltpu.VMEM((nper,), jnp.int32),
          pltpu.VMEM((2, chunk, _GD), jnp.float32),
          pltpu.SemaphoreType.DMA,
          pltpu.SemaphoreType.DMA,
      ],
  )(functools.partial(_gather_body, nch, chunk))
  return f(p2, ids2)


_BB = 128
_LL = 40


def _finish_body(g_ref, ids_ref, pos_ref, *rest):
  o_ref = rest[-1]
  eye = (lax.broadcasted_iota(jnp.int32, (_BB, _BB), 0) ==
         lax.broadcasted_iota(jnp.int32, (_BB, _BB), 1)).astype(jnp.float32)
  dn = (((0,), (0,)), ((), ()))
  yts = []
  for li in range(_LL):
    gl = g_ref[:, li, :]
    yts.append(lax.dot_general(gl, eye, dn, preferred_element_type=jnp.float32))
  yt = jnp.stack(yts, axis=0)
  par = (ids_ref[...] >= _D)[:, None, :]
  j = lax.broadcasted_iota(jnp.int32, (_LL, _GD, _BB), 1)
  ym = jnp.where((j >= _ED) == par, yt, 0.0)
  yv = ym[:, :_ED, :] + ym[:, _ED:, :]
  o_ref[...] = yv + pos_ref[...]


def _finish(g3, ids_t, pos3, b, l, b_off, alias_in=None):
  bs = g3.shape[0]
  grid = (l // _LL, bs // _BB)
  ins = [g3, ids_t, pos3]
  in_specs = [
      pl.BlockSpec((_BB, _LL, _GD), lambda il, ib: (ib, il, 0)),
      pl.BlockSpec((_LL, _BB), lambda il, ib: (il, ib + b_off)),
      pl.BlockSpec((_LL, _LD, 1), lambda il, ib: (il, 0, 0)),
  ]
  kwargs = {}
  if alias_in is not None:
    ins.append(alias_in)
    in_specs.append(pl.BlockSpec(memory_space=pl.ANY))
    kwargs["input_output_aliases"] = {3: 0}
  return pl.pallas_call(
      _finish_body,
      grid=grid,
      in_specs=in_specs,
      out_specs=pl.BlockSpec((_LL, _LD, _BB), lambda il, ib: (il, 0, ib + b_off)),
      out_shape=jax.ShapeDtypeStruct((l, _LD, b), jnp.float32),
      **kwargs,
  )(*ins)


def kernel(token_ids, embed_table, proj_w, proj_b, pos_embed):
  b, l = token_ids.shape
  n = b * l
  assert n % (_NW * _CHUNK) == 0
  nch = n // (_NW * _CHUNK)
  ids = token_ids.astype(jnp.int32)
  table_t = embed_table.T
  z = jnp.zeros((_ED, _ED), jnp.float32)
  wd = jnp.concatenate([jnp.concatenate([proj_w, z], 1),
                        jnp.concatenate([z, proj_w], 1)], 0)
  p2 = _project(table_t, wd)
  phys = jnp.where(ids < _D, ids, ids - _D)
  ids_t = ids.T
  pos_bt = jnp.transpose(pos_embed, (0, 2, 1))[0]
  pos3 = (pos_bt[:, :l].T + proj_b[None, :])[:, :, None]
  nsplit = 4
  chunk = 64
  bh = b // nsplit
  out_t = None
  for h in range(nsplit):
    ph = phys[h * bh:(h + 1) * bh].reshape(_NW, bh * l // _NW)
    g3 = _sc_gather(p2, ph, chunk).reshape(bh, l, _GD)
    out_t = _finish(g3, ids_t, pos3, b, l, h * (bh // _BB), alias_in=out_t)
  return jnp.transpose(out_t, (2, 0, 1))


# speedup vs baseline: 1.0170x; 1.0170x over previous
import functools

import jax
import jax.numpy as jnp
from jax import lax
from jax.experimental import pallas as pl
from jax.experimental.pallas import tpu as pltpu
from jax.experimental.pallas import tpu_sc as plsc

_ED = 64
_LD = 64
_GD = 2 * _ED
_S = 16384
_NB = 31
_D = _NB * _S
_V = 1000000
_CHUNK = 128
_NC = 2
_NS = 16
_NW = _NC * _NS


def _project_body(alo_ref, ahi_ref, wd_ref, o_ref):
  i = pl.program_id(0)
  lane_g = lax.broadcasted_iota(jnp.int32, (_ED, _S), 1) + i * _S
  ahi = jnp.where(lane_g >= _V - _D, 0.0, ahi_ref[...])
  a2 = jnp.concatenate([alo_ref[...], ahi], axis=0)
  dn = (((0,), (0,)), ((), ()))
  o_ref[...] = lax.dot_general(a2, wd_ref[...], dn,
                               preferred_element_type=jnp.float32)


def _project(table_t, wd):
  return pl.pallas_call(
      _project_body,
      grid=(_NB,),
      in_specs=[
          pl.BlockSpec((_ED, _S), lambda i: (0, i)),
          pl.BlockSpec((_ED, _S), lambda i: (0, jnp.minimum(i + _NB, _V // _S))),
          pl.BlockSpec((_GD, _GD), lambda i: (0, 0)),
      ],
      out_specs=pl.BlockSpec((_S, _GD), lambda i: (i, 0)),
      out_shape=jax.ShapeDtypeStruct((_D, _GD), jnp.float32),
  )(table_t, table_t, wd)


def _gather_body(nch, chunk, table_hbm, idx_hbm, out_hbm, idx_v, rows_v,
                 sem0, sem1):
  wid = lax.axis_index("s") * _NC + lax.axis_index("c")
  base = wid * (nch * chunk)
  pltpu.sync_copy(idx_hbm.at[wid], idx_v)

  sems = (sem0, sem1)

  def start(c, buf):
    pltpu.make_async_copy(
        table_hbm.at[idx_v.at[pl.ds(c * chunk, chunk)]],
        rows_v.at[buf], sems[buf]).start()

  def wait(c, buf):
    pltpu.make_async_copy(
        table_hbm.at[idx_v.at[pl.ds(c * chunk, chunk)]],
        rows_v.at[buf], sems[buf]).wait()

  def store(c, buf):
    pltpu.sync_copy(rows_v.at[buf], out_hbm.at[pl.ds(base + c * chunk, chunk)])

  start(0, 0)

  def body(g, carry):
    c = 2 * g
    start(c + 1, 1)
    wait(c, 0)
    store(c, 0)

    @pl.when(c + 2 < nch)
    def _():
      start(c + 2, 0)

    wait(c + 1, 1)
    store(c + 1, 1)
    return carry

  lax.fori_loop(0, nch // 2, body, 0)
  if nch % 2 == 1:
    wait(nch - 1, 0)
    store(nch - 1, 0)


def _sc_gather(p2, ids2, chunk):
  nw, nper = ids2.shape
  nch = nper // chunk
  n = nw * nper
  mesh = plsc.VectorSubcoreMesh(core_axis_name="c", subcore_axis_name="s")
  f = functools.partial(
      pl.kernel,
      mesh=mesh,
      out_type=jax.ShapeDtypeStruct((n, _GD), jnp.float32),
      scratch_types=[
          pltpu.VMEM((nper,), jnp.int32),
          pltpu.VMEM((2, chunk, _GD), jnp.float32),
          pltpu.SemaphoreType.DMA,
          pltpu.SemaphoreType.DMA,
      ],
  )(functools.partial(_gather_body, nch, chunk))
  return f(p2, ids2)


_BB = 128
_LL = 40


def _finish_body(g_ref, ids_ref, pos_ref, *rest):
  o_ref = rest[-1]
  eye = (lax.broadcasted_iota(jnp.int32, (_BB, _BB), 0) ==
         lax.broadcasted_iota(jnp.int32, (_BB, _BB), 1)).astype(jnp.float32)
  dn = (((0,), (0,)), ((), ()))
  yts = []
  for li in range(_LL):
    gl = g_ref[:, li, :]
    yts.append(lax.dot_general(gl, eye, dn, preferred_element_type=jnp.float32))
  yt = jnp.stack(yts, axis=0)
  par = (ids_ref[...] >= _D)[:, None, :]
  j = lax.broadcasted_iota(jnp.int32, (_LL, _GD, _BB), 1)
  ym = jnp.where((j >= _ED) == par, yt, 0.0)
  yv = ym[:, :_ED, :] + ym[:, _ED:, :]
  o_ref[...] = yv + pos_ref[...]


def _finish(g3, ids_t, pos3, b, l, b_off, alias_in=None):
  bs = g3.shape[0]
  grid = (l // _LL, bs // _BB)
  ins = [g3, ids_t, pos3]
  in_specs = [
      pl.BlockSpec((_BB, _LL, _GD), lambda il, ib: (ib, il, 0)),
      pl.BlockSpec((_LL, _BB), lambda il, ib: (il, ib + b_off)),
      pl.BlockSpec((_LL, _LD, 1), lambda il, ib: (il, 0, 0)),
  ]
  kwargs = {}
  if alias_in is not None:
    ins.append(alias_in)
    in_specs.append(pl.BlockSpec(memory_space=pl.ANY))
    kwargs["input_output_aliases"] = {3: 0}
  return pl.pallas_call(
      _finish_body,
      grid=grid,
      in_specs=in_specs,
      out_specs=pl.BlockSpec((_LL, _LD, _BB), lambda il, ib: (il, 0, ib + b_off)),
      out_shape=jax.ShapeDtypeStruct((l, _LD, b), jnp.float32),
      **kwargs,
  )(*ins)


def kernel(token_ids, embed_table, proj_w, proj_b, pos_embed):
  b, l = token_ids.shape
  n = b * l
  assert n % (_NW * _CHUNK) == 0
  nch = n // (_NW * _CHUNK)
  ids = token_ids.astype(jnp.int32)
  table_t = embed_table.T
  z = jnp.zeros((_ED, _ED), jnp.float32)
  wd = jnp.concatenate([jnp.concatenate([proj_w, z], 1),
                        jnp.concatenate([z, proj_w], 1)], 0)
  p2 = _project(table_t, wd)
  phys = jnp.where(ids < _D, ids, ids - _D)
  ids_t = ids.T
  pos_bt = jnp.transpose(pos_embed, (0, 2, 1))[0]
  pos3 = (pos_bt[:, :l].T + proj_b[None, :])[:, :, None]
  nsplit = 2
  chunk = _CHUNK
  bh = b // nsplit
  out_t = None
  for h in range(nsplit):
    ph = phys[h * bh:(h + 1) * bh].reshape(_NW, bh * l // _NW)
    g3 = _sc_gather(p2, ph, chunk).reshape(bh, l, _GD)
    out_t = _finish(g3, ids_t, pos3, b, l, h * (bh // _BB), alias_in=out_t)
  return jnp.transpose(out_t, (2, 0, 1))
